# baseline (speedup 1.0000x reference)
import jax
import jax.numpy as jnp
from jax.experimental import pallas as pl
from jax.experimental.pallas import tpu as pltpu

_DILATIONS = (1, 2, 4, 8, 16)
_ND = len(_DILATIONS)
_BN_EPS = 1e-3
_HALO = 16
_HB = 16
_W = 256
_WH = _W // 2
_WS = _WH + 2 * _HALO

_INV_SQRT2 = 0.7071067811865476


def _body(xa_ref, xb_ref, xc_ref, kr_ref, sc_ref, sh_ref,
          o0, o1, o2, o3, o4, pt_ref, acc_ref):
    outs = (o0, o1, o2, o3, o4)
    xa = xa_ref[0].reshape(_HB * _WS, 128)
    xb = xb_ref[0].reshape(_HB * _WS, 128)
    xc = xc_ref[0].reshape(_HB * _WS, 128)
    for t in range(9):
        kt = kr_ref[t]
        pa = jnp.dot(xa, kt, preferred_element_type=jnp.float32)
        pb = jnp.dot(xb, kt, preferred_element_type=jnp.float32)
        pc = jnp.dot(xc, kt, preferred_element_type=jnp.float32)
        pt_ref[0:_HB] = pa.reshape(_HB, _WS, 128)
        pt_ref[_HB:2 * _HB] = pb.reshape(_HB, _WS, 128)
        pt_ref[2 * _HB:3 * _HB] = pc.reshape(_HB, _WS, 128)
        ky, kx = divmod(t, 3)
        for di, d in enumerate(_DILATIONS):
            r0 = _HALO + d * (ky - 1)
            c0 = _HALO + d * (kx - 1)
            win = pt_ref[r0:r0 + _HB, c0:c0 + _WH, :]
            if t == 0:
                acc_ref[di] = win
            else:
                acc_ref[di] += win
    for di in range(_ND):
        for r in range(0, _HB, 8):
            y = acc_ref[di, r:r + 8] * sc_ref[di] + sh_ref[di]
            g = 0.5 * y * (1.0 + jax.lax.erf(y * _INV_SQRT2))
            outs[di][0, r:r + 8, 0:_WH, :] = g[:, :, 0:64]
            outs[di][0, r:r + 8, _WH:_W, :] = g[:, :, 64:128]


def _repack_body(x_ref, o_ref):
    i = pl.program_id(1)
    xin = x_ref[0]
    v = xin.astype(jnp.bfloat16)
    o_ref[0, :, 0:_HALO, 0:64] = jnp.zeros((_HB, _HALO, 64), jnp.bfloat16)
    o_ref[0, :, _HALO:_WS, 0:64] = v[:, 0:_WS - _HALO, :]
    o_ref[0, :, 0:_WS - _HALO, 64:128] = v[:, _WH - _HALO:_W, :]
    o_ref[0, :, _WS - _HALO:_WS, 64:128] = jnp.zeros((_HB, _HALO, 64), jnp.bfloat16)

    @pl.when((i == 0) | (i == (_W + 2 * _HALO) // _HB - 1))
    def _():
        o_ref[...] = jnp.zeros_like(o_ref)


def _repack(x):
    B, H, W, C = x.shape
    nhp = (H + 2 * _HALO) // _HB
    return pl.pallas_call(
        _repack_body,
        grid=(B, nhp),
        in_specs=[pl.BlockSpec((1, _HB, W, C),
                               lambda b, i: (b, jnp.clip(i - 1, 0, H // _HB - 1), 0, 0))],
        out_specs=pl.BlockSpec((1, _HB, _WS, 2 * C), lambda b, i: (b, i, 0, 0)),
        out_shape=jax.ShapeDtypeStruct((B, H + 2 * _HALO, _WS, 2 * C), jnp.bfloat16),
        compiler_params=pltpu.CompilerParams(
            dimension_semantics=("parallel", "arbitrary"),
        ),
        name="repack_pad_pair",
    )(x)


def kernel(x, kernel, gamma, beta, mov_mean, mov_var):
    B, H, W, C = x.shape
    scale = gamma * jax.lax.rsqrt(mov_var + _BN_EPS)
    shift = beta - mov_mean * scale
    sc2 = jnp.concatenate([scale, scale], axis=-1)
    sh2 = jnp.concatenate([shift, shift], axis=-1)
    xp = _repack(x)
    kb = kernel.reshape(9, C, C).astype(jnp.bfloat16)
    z = jnp.zeros((9, C, C), jnp.bfloat16)
    kr = jnp.concatenate(
        [jnp.concatenate([kb, z], -1), jnp.concatenate([z, kb], -1)], axis=1)

    nh = H // _HB
    grid = (B, nh)
    blk_in = (1, _HB, _WS, 2 * C)
    out_sds = jax.ShapeDtypeStruct((B, H, W, C), jnp.float32)
    out_spec = pl.BlockSpec((1, _HB, W, C), lambda b, i: (b, i, 0, 0))

    outs = pl.pallas_call(
        _body,
        grid=grid,
        in_specs=[
            pl.BlockSpec(blk_in, lambda b, i: (b, i, 0, 0)),
            pl.BlockSpec(blk_in, lambda b, i: (b, i + 1, 0, 0)),
            pl.BlockSpec(blk_in, lambda b, i: (b, i + 2, 0, 0)),
            pl.BlockSpec((9, 2 * C, 2 * C), lambda b, i: (0, 0, 0)),
            pl.BlockSpec((_ND, 2 * C), lambda b, i: (0, 0)),
            pl.BlockSpec((_ND, 2 * C), lambda b, i: (0, 0)),
        ],
        out_specs=[out_spec] * _ND,
        out_shape=[out_sds] * _ND,
        scratch_shapes=[
            pltpu.VMEM((3 * _HB, _WS, 2 * C), jnp.float32),
            pltpu.VMEM((_ND, _HB, _WH, 2 * C), jnp.float32),
        ],
        compiler_params=pltpu.CompilerParams(
            dimension_semantics=("parallel", "arbitrary"),
            vmem_limit_bytes=56 * 1024 * 1024,
        ),
        name="shared_tap_dilated_conv",
    )(xp, xp, xp, kr, sc2, sh2)
    return tuple(outs)


# speedup vs baseline: 1.3446x; 1.3446x over previous
import jax
import jax.numpy as jnp
from jax.experimental import pallas as pl
from jax.experimental.pallas import tpu as pltpu

_DILATIONS = (1, 2, 4, 8, 16)
_ND = len(_DILATIONS)
_BN_EPS = 1e-3
_HALO = 16
_HB = 16
_W = 256
_WH = _W // 2
_WS = _WH + 2 * _HALO

_INV_SQRT2 = 0.7071067811865476


def _body(xa_ref, xb_ref, xc_ref, kr_ref, sc_ref, sh_ref,
          o0, o1, o2, o3, o4, pt_ref, acc_ref):
    outs = (o0, o1, o2, o3, o4)
    xa = xa_ref[0].reshape(_HB * _WS, 128)
    xb = xb_ref[0].reshape(_HB * _WS, 128)
    xc = xc_ref[0].reshape(_HB * _WS, 128)
    for t in range(9):
        kt = kr_ref[t]
        pa = jnp.dot(xa, kt, preferred_element_type=jnp.float32)
        pb = jnp.dot(xb, kt, preferred_element_type=jnp.float32)
        pc = jnp.dot(xc, kt, preferred_element_type=jnp.float32)
        pt_ref[0:_HB] = pa.reshape(_HB, _WS, 128)
        pt_ref[_HB:2 * _HB] = pb.reshape(_HB, _WS, 128)
        pt_ref[2 * _HB:3 * _HB] = pc.reshape(_HB, _WS, 128)
        ky, kx = divmod(t, 3)
        for di, d in enumerate(_DILATIONS):
            r0 = _HALO + d * (ky - 1)
            c0 = _HALO + d * (kx - 1)
            win = pt_ref[r0:r0 + _HB, c0:c0 + _WH, :]
            if t == 0:
                acc_ref[di] = win
            else:
                acc_ref[di] += win
    for di in range(_ND):
        for r in range(0, _HB, 8):
            y = acc_ref[di, r:r + 8] * sc_ref[di] + sh_ref[di]
            g = 0.5 * y * (1.0 + jax.lax.erf(y * _INV_SQRT2))
            outs[di][0, r:r + 8, :, 0:_WH] = jnp.swapaxes(g[:, :, 0:64], 1, 2)
            outs[di][0, r:r + 8, :, _WH:_W] = jnp.swapaxes(g[:, :, 64:128], 1, 2)


def _repack_body(x_ref, o_ref):
    i = pl.program_id(1)
    xt = jnp.swapaxes(x_ref[0], 1, 2).astype(jnp.bfloat16)
    o_ref[0, :, 0:_HALO, 0:64] = jnp.zeros((_HB, _HALO, 64), jnp.bfloat16)
    o_ref[0, :, _HALO:_WS, 0:64] = xt[:, 0:_WS - _HALO, :]
    o_ref[0, :, 0:_WS - _HALO, 64:128] = xt[:, _WH - _HALO:_W, :]
    o_ref[0, :, _WS - _HALO:_WS, 64:128] = jnp.zeros((_HB, _HALO, 64), jnp.bfloat16)

    @pl.when((i == 0) | (i == (_W + 2 * _HALO) // _HB - 1))
    def _():
        o_ref[...] = jnp.zeros_like(o_ref)


def _repack(xt):
    B, H, C, W = xt.shape
    nhp = (H + 2 * _HALO) // _HB
    return pl.pallas_call(
        _repack_body,
        grid=(B, nhp),
        in_specs=[pl.BlockSpec((1, _HB, C, W),
                               lambda b, i: (b, jnp.clip(i - 1, 0, H // _HB - 1), 0, 0))],
        out_specs=pl.BlockSpec((1, _HB, _WS, 2 * C), lambda b, i: (b, i, 0, 0)),
        out_shape=jax.ShapeDtypeStruct((B, H + 2 * _HALO, _WS, 2 * C), jnp.bfloat16),
        compiler_params=pltpu.CompilerParams(
            dimension_semantics=("parallel", "arbitrary"),
        ),
        name="repack_pad_pair",
    )(xt)


def kernel(x, kernel, gamma, beta, mov_mean, mov_var):
    B, H, W, C = x.shape
    scale = gamma * jax.lax.rsqrt(mov_var + _BN_EPS)
    shift = beta - mov_mean * scale
    sc2 = jnp.concatenate([scale, scale], axis=-1)
    sh2 = jnp.concatenate([shift, shift], axis=-1)
    xt = jnp.transpose(x, (0, 1, 3, 2))
    xp = _repack(xt)
    kb = kernel.reshape(9, C, C).astype(jnp.bfloat16)
    z = jnp.zeros((9, C, C), jnp.bfloat16)
    kr = jnp.concatenate(
        [jnp.concatenate([kb, z], -1), jnp.concatenate([z, kb], -1)], axis=1)

    nh = H // _HB
    grid = (B, nh)
    blk_in = (1, _HB, _WS, 2 * C)
    out_sds = jax.ShapeDtypeStruct((B, H, C, W), jnp.float32)
    out_spec = pl.BlockSpec((1, _HB, C, W), lambda b, i: (b, i, 0, 0))

    outs = pl.pallas_call(
        _body,
        grid=grid,
        in_specs=[
            pl.BlockSpec(blk_in, lambda b, i: (b, i, 0, 0)),
            pl.BlockSpec(blk_in, lambda b, i: (b, i + 1, 0, 0)),
            pl.BlockSpec(blk_in, lambda b, i: (b, i + 2, 0, 0)),
            pl.BlockSpec((9, 2 * C, 2 * C), lambda b, i: (0, 0, 0)),
            pl.BlockSpec((_ND, 2 * C), lambda b, i: (0, 0)),
            pl.BlockSpec((_ND, 2 * C), lambda b, i: (0, 0)),
        ],
        out_specs=[out_spec] * _ND,
        out_shape=[out_sds] * _ND,
        scratch_shapes=[
            pltpu.VMEM((3 * _HB, _WS, 2 * C), jnp.float32),
            pltpu.VMEM((_ND, _HB, _WH, 2 * C), jnp.float32),
        ],
        compiler_params=pltpu.CompilerParams(
            dimension_semantics=("parallel", "arbitrary"),
            vmem_limit_bytes=56 * 1024 * 1024,
        ),
        name="shared_tap_dilated_conv",
    )(xp, xp, xp, kr, sc2, sh2)
    return tuple(jnp.transpose(o, (0, 1, 3, 2)) for o in outs)
